# baseline (speedup 1.0000x reference)
import functools

import jax
import jax.numpy as jnp
from jax import lax
from jax.experimental import pallas as pl
from jax.experimental.pallas import tpu as pltpu, tpu_sc as plsc

NUM_CLS = 1000
N_ROWS = 4096 * 50
NW = 32
ROWS_PER_W = N_ROWS // NW
R = 32
NBUF = 2
CHUNKS = ROWS_PER_W // R
BUF_WORDS = R * NUM_CLS

_mesh = plsc.VectorSubcoreMesh(core_axis_name="c", subcore_axis_name="s")


@functools.partial(
    pl.kernel,
    out_type=jax.ShapeDtypeStruct((N_ROWS * NUM_CLS,), jnp.float32),
    mesh=_mesh,
    scratch_types=[
        pltpu.VMEM((ROWS_PER_W,), jnp.int32),
        pltpu.VMEM((NBUF * BUF_WORDS,), jnp.float32),
        pltpu.SemaphoreType.DMA,
        pltpu.SemaphoreType.DMA,
    ],
    compiler_params=pltpu.CompilerParams(needs_layout_passes=False),
)
def _onehot_sc(x_hbm, out_hbm, idx_v, buf_v, sem0, sem1):
    wid = lax.axis_index("s") * 2 + lax.axis_index("c")
    base_row = wid * ROWS_PER_W
    sems = (sem0, sem1)

    pltpu.sync_copy(x_hbm.at[pl.ds(base_row, ROWS_PER_W)], idx_v)

    zeros16 = jnp.zeros((16,), jnp.float32)
    ones16 = jnp.ones((16,), jnp.float32)
    row_off16 = lax.iota(jnp.int32, 16) * NUM_CLS

    def _zero(i, carry):
        base = i * 256
        for u in range(16):
            buf_v[pl.ds(base + u * 16, 16)] = zeros16
        return carry

    lax.fori_loop(0, (NBUF * BUF_WORDS) // 256, _zero, 0)

    def set_chunk(c, slot, vals16):
        for k in range(R // 16):
            cols = idx_v[pl.ds(c * R + k * 16, 16)]
            pos = (slot * BUF_WORDS + k * 16 * NUM_CLS) + row_off16 + cols
            plsc.store_scatter(buf_v, [pos], vals16)

    def dma(slot, c):
        flat = (base_row + c * R) * NUM_CLS
        return pltpu.make_async_copy(
            buf_v.at[pl.ds(slot * BUF_WORDS, BUF_WORDS)],
            out_hbm.at[pl.ds(flat, BUF_WORDS)],
            sems[slot],
        )

    for b in range(NBUF):
        set_chunk(b, b, ones16)
        dma(b, b).start()

    def step(t, carry):
        g = t * NBUF
        for b in range(NBUF):
            c = g + b
            dma(b, c).wait()
            set_chunk(c - NBUF, b, zeros16)
            set_chunk(c, b, ones16)
            dma(b, c).start()
        return carry

    lax.fori_loop(1, CHUNKS // NBUF, step, 0)

    for b in range(NBUF):
        dma(b, CHUNKS - NBUF + b).wait()


def kernel(x):
    x_flat = x.astype(jnp.int32).reshape(N_ROWS)
    out = _onehot_sc(x_flat)
    return out.reshape(4096, 50, NUM_CLS)


# speedup vs baseline: 1.9805x; 1.9805x over previous
import functools

import jax
import jax.numpy as jnp
from jax import lax
from jax.experimental import pallas as pl
from jax.experimental.pallas import tpu as pltpu, tpu_sc as plsc

NUM_CLS = 1000
NI, NJ = 4096, 50
NW = 32
I_PER_W = NI // NW
NBUF = 2

_mesh = plsc.VectorSubcoreMesh(core_axis_name="c", subcore_axis_name="s")


@functools.partial(
    pl.kernel,
    out_type=jax.ShapeDtypeStruct((NI, NJ, NUM_CLS), jnp.float32),
    mesh=_mesh,
    scratch_types=[
        pltpu.VMEM((I_PER_W * NJ + 16,), jnp.int32),
        pltpu.VMEM((NBUF, NJ, NUM_CLS), jnp.float32),
        pltpu.SemaphoreType.DMA,
        pltpu.SemaphoreType.DMA,
        pltpu.SemaphoreType.DMA,
    ],
    compiler_params=pltpu.CompilerParams(needs_layout_passes=False),
)
def _onehot_sc(x_hbm, zeros_hbm, out_hbm, idx_v, buf_v, sem0, sem1, semz):
    wid = lax.axis_index("s") * 2 + lax.axis_index("c")
    base_i = wid * I_PER_W
    sems = (sem0, sem1)

    idx_cp = pltpu.make_async_copy(
        x_hbm.at[pl.ds(base_i * NJ, I_PER_W * NJ)],
        idx_v.at[pl.ds(0, I_PER_W * NJ)],
        semz,
    )
    idx_cp.start()
    zcp = pltpu.make_async_copy(zeros_hbm, buf_v, sems[0])
    zcp.start()
    idx_cp.wait()
    zcp.wait()

    iota16 = lax.iota(jnp.int32, 16)
    zeros16 = jnp.zeros((16,), jnp.float32)
    ones16 = jnp.ones((16,), jnp.float32)
    ngrp = (NJ + 15) // 16
    masks = [(k * 16 + iota16) < NJ for k in range(ngrp)]

    def set_chunk(c, slot, vals16):
        slot16 = jnp.full((16,), slot, jnp.int32)
        for k in range(ngrp):
            cols = idx_v[pl.ds(c * NJ + k * 16, 16)]
            j16 = k * 16 + iota16
            if (k + 1) * 16 <= NJ:
                plsc.store_scatter(buf_v, [slot16, j16, cols], vals16)
            else:
                plsc.store_scatter(buf_v, [slot16, j16, cols], vals16,
                                   mask=masks[k])

    def dma(slot, c):
        return pltpu.make_async_copy(
            buf_v.at[slot], out_hbm.at[base_i + c], sems[slot])

    for b in range(NBUF):
        set_chunk(b, b, ones16)
        dma(b, b).start()

    def step(t, carry):
        g = t * NBUF
        for b in range(NBUF):
            c = g + b
            dma(b, c).wait()
            set_chunk(c - NBUF, b, zeros16)
            set_chunk(c, b, ones16)
            dma(b, c).start()
        return carry

    lax.fori_loop(1, I_PER_W // NBUF, step, 0)

    for b in range(NBUF):
        dma(b, I_PER_W - NBUF + b).wait()


def kernel(x):
    x_flat = x.astype(jnp.int32).reshape(NI * NJ)
    zeros = jnp.zeros((NBUF, NJ, NUM_CLS), jnp.float32)
    return _onehot_sc(x_flat, zeros)


# speedup vs baseline: 7.7188x; 3.8974x over previous
import functools

import jax
import jax.numpy as jnp
from jax import lax
from jax.experimental import pallas as pl
from jax.experimental.pallas import tpu as pltpu, tpu_sc as plsc

NUM_CLS = 1000
NI, NJ = 4096, 50
NW = 32
IW = NI // NW
CW = 200
NCH = NUM_CLS // CW
CHUNKS = NJ * NCH
NBUF = 2

_mesh = plsc.VectorSubcoreMesh(core_axis_name="c", subcore_axis_name="s")


@functools.partial(
    pl.kernel,
    out_type=jax.ShapeDtypeStruct((NJ, NUM_CLS, NI), jnp.float32),
    mesh=_mesh,
    scratch_types=[
        pltpu.VMEM((NJ, IW), jnp.int32),
        pltpu.VMEM((NBUF, CW, IW), jnp.float32),
        pltpu.SemaphoreType.DMA,
        pltpu.SemaphoreType.DMA,
        pltpu.SemaphoreType.DMA,
    ],
    compiler_params=pltpu.CompilerParams(needs_layout_passes=False),
)
def _onehot_sc(xt_hbm, zeros_hbm, out_hbm, idx_v, buf_v, sem0, sem1, semz):
    wid = lax.axis_index("s") * 2 + lax.axis_index("c")
    i_base = wid * IW
    sems = (sem0, sem1)

    idx_cp = pltpu.make_async_copy(
        xt_hbm.at[:, pl.ds(i_base, IW)], idx_v, semz)
    idx_cp.start()
    zcp = pltpu.make_async_copy(zeros_hbm, buf_v, sems[0])
    zcp.start()
    idx_cp.wait()
    zcp.wait()

    iota16 = lax.iota(jnp.int32, 16)
    zeros16 = jnp.zeros((16,), jnp.float32)
    ones16 = jnp.ones((16,), jnp.float32)

    def set_chunk(m, slot, vals16):
        jj = m // NCH
        c0 = (m % NCH) * CW
        slot16 = jnp.full((16,), slot, jnp.int32)
        for g in range(IW // 16):
            cols = idx_v[jj, pl.ds(g * 16, 16)]
            rel = cols - c0
            mask = (rel >= 0) & (rel < CW)
            i16 = g * 16 + iota16
            plsc.store_scatter(buf_v, [slot16, rel, i16], vals16, mask=mask)

    def dma(slot, m):
        jj = m // NCH
        c0 = (m % NCH) * CW
        return pltpu.make_async_copy(
            buf_v.at[slot],
            out_hbm.at[jj, pl.ds(c0, CW), pl.ds(i_base, IW)],
            sems[slot],
        )

    for b in range(NBUF):
        set_chunk(b, b, ones16)
        dma(b, b).start()

    def step(t, carry):
        g = t * NBUF
        for b in range(NBUF):
            m = g + b
            dma(b, m).wait()
            set_chunk(m - NBUF, b, zeros16)
            set_chunk(m, b, ones16)
            dma(b, m).start()
        return carry

    lax.fori_loop(1, CHUNKS // NBUF, step, 0)

    for b in range(NBUF):
        dma(b, CHUNKS - NBUF + b).wait()


def kernel(x):
    xt = x.astype(jnp.int32).T
    zeros = jnp.zeros((NBUF, CW, IW), jnp.float32)
    out = _onehot_sc(xt, zeros)
    return out.transpose(2, 0, 1)


# speedup vs baseline: 7.7312x; 1.0016x over previous
import functools

import jax
import jax.numpy as jnp
from jax import lax
from jax.experimental import pallas as pl
from jax.experimental.pallas import tpu as pltpu, tpu_sc as plsc

NUM_CLS = 1000
NI, NJ = 4096, 50
NW = 32
NIS = 16
IW = NI // NIS
JW = NJ // 2
CW = 200
NCH = NUM_CLS // CW
CHUNKS = JW * NCH
NBUF = 2

_mesh = plsc.VectorSubcoreMesh(core_axis_name="c", subcore_axis_name="s")


@functools.partial(
    pl.kernel,
    out_type=jax.ShapeDtypeStruct((NJ, NUM_CLS, NI), jnp.float32),
    mesh=_mesh,
    scratch_types=[
        pltpu.VMEM((JW * IW,), jnp.int32),
        pltpu.VMEM((NBUF, CW, IW), jnp.float32),
        pltpu.SemaphoreType.DMA,
        pltpu.SemaphoreType.DMA,
        pltpu.SemaphoreType.DMA,
    ],
    compiler_params=pltpu.CompilerParams(needs_layout_passes=False),
)
def _onehot_sc(xt_hbm, zeros_hbm, out_hbm, idx_v, buf_v, sem0, sem1, semz):
    wid = lax.axis_index("s") * 2 + lax.axis_index("c")
    i_base = (wid % NIS) * IW
    j_base = (wid // NIS) * JW
    sems = (sem0, sem1)

    idx_cps = [
        pltpu.make_async_copy(
            xt_hbm.at[pl.ds((j_base + jj) * NI + i_base, IW)],
            idx_v.at[pl.ds(jj * IW, IW)], semz)
        for jj in range(JW)
    ]
    for cp in idx_cps:
        cp.start()
    zcp = pltpu.make_async_copy(zeros_hbm, buf_v, sems[0])
    zcp.start()
    for cp in idx_cps:
        cp.wait()
    zcp.wait()

    iota16 = lax.iota(jnp.int32, 16)
    zeros16 = jnp.zeros((16,), jnp.float32)
    ones16 = jnp.ones((16,), jnp.float32)

    def set_chunk(m, slot, vals16):
        jj = m // NCH
        c0 = (m % NCH) * CW
        slot16 = jnp.full((16,), slot, jnp.int32)
        for g in range(IW // 16):
            cols = idx_v[pl.ds(jj * IW + g * 16, 16)]
            rel = cols - c0
            mask = (rel >= 0) & (rel < CW)
            i16 = g * 16 + iota16
            plsc.store_scatter(buf_v, [slot16, rel, i16], vals16, mask=mask)

    def dma(slot, m):
        jj = j_base + m // NCH
        c0 = (m % NCH) * CW
        return pltpu.make_async_copy(
            buf_v.at[slot],
            out_hbm.at[jj, pl.ds(c0, CW), pl.ds(i_base, IW)],
            sems[slot],
        )

    for b in range(NBUF):
        set_chunk(b, b, ones16)
        dma(b, b).start()

    def step(t, carry):
        g = t * NBUF
        for b in range(NBUF):
            m = g + b
            dma(b, m).wait()
            set_chunk(m - NBUF, b, zeros16)
            set_chunk(m, b, ones16)
            dma(b, m).start()
        return carry

    lax.fori_loop(1, CHUNKS // NBUF, step, 0)

    for m in range(NBUF * (CHUNKS // NBUF), CHUNKS):
        b = m % NBUF
        dma(b, m - NBUF).wait()
        set_chunk(m - NBUF, b, zeros16)
        set_chunk(m, b, ones16)
        dma(b, m).start()

    for b in range(NBUF):
        dma(b, CHUNKS - NBUF + b).wait()


def kernel(x):
    xt = x.astype(jnp.int32).T.reshape(NJ * NI)
    zeros = jnp.zeros((NBUF, CW, IW), jnp.float32)
    out = _onehot_sc(xt, zeros)
    return out.transpose(2, 0, 1)
